# baseline (speedup 1.0000x reference)
import functools

import jax
import jax.numpy as jnp
from jax import lax
from jax.experimental import pallas as pl
from jax.experimental.pallas import tpu as pltpu
from jax.experimental.pallas import tpu_sc as plsc

B = 4096
F = 26
D = 32
NW = 32
BPW = B // NW
ROWS = BPW * F
NCHUNK = ROWS // BPW
OUTW = D + 1
LANES = 16


def _build():
    mesh = plsc.VectorSubcoreMesh(core_axis_name="c", subcore_axis_name="s")

    @functools.partial(
        pl.kernel,
        out_type=jax.ShapeDtypeStruct((B * OUTW,), jnp.float32),
        mesh=mesh,
        compiler_params=pltpu.CompilerParams(use_tc_tiling_on_sc=False),
        scratch_types=[
            pltpu.VMEM((NCHUNK, BPW), jnp.int32),
            pltpu.VMEM((ROWS, D), jnp.float32),
            pltpu.VMEM((BPW * OUTW,), jnp.float32),
            pltpu.SemaphoreType.DMA,
        ],
    )
    def sc_kernel(idx_hbm, table_hbm, out_hbm, idx_v, rows_v, out_v, sem):
        wid = lax.axis_index("s") * 2 + lax.axis_index("c")

        pltpu.sync_copy(idx_hbm.at[wid], idx_v)

        copies = [
            pltpu.async_copy(
                table_hbm.at[idx_v.at[j]],
                rows_v.at[pl.ds(j * BPW, BPW)],
                sem,
            )
            for j in range(NCHUNK)
        ]

        for cp in copies:
            cp.wait()

        def pool_row(b, carry):
            r = b * F
            acc_lo = rows_v[r, pl.ds(0, LANES)]
            acc_hi = rows_v[r, pl.ds(LANES, LANES)]
            for f in range(1, F):
                acc_lo = acc_lo + rows_v[r + f, pl.ds(0, LANES)]
                acc_hi = acc_hi + rows_v[r + f, pl.ds(LANES, LANES)]
            o = b * OUTW
            out_v[pl.ds(o, LANES)] = jnp.ones((LANES,), jnp.float32)
            out_v[pl.ds(o + 1, LANES)] = acc_lo
            out_v[pl.ds(o + 1 + LANES, LANES)] = acc_hi
            return carry

        lax.fori_loop(0, BPW, pool_row, 0)

        pltpu.sync_copy(
            out_v, out_hbm.at[pl.ds(wid * (BPW * OUTW), BPW * OUTW)]
        )

    return sc_kernel


_SC_KERNEL = _build()


@jax.jit
def kernel(user_feature_ids, embedding_weight):
    idx = jnp.asarray(user_feature_ids, jnp.int32).reshape(NW, NCHUNK, BPW)
    flat = _SC_KERNEL(idx, embedding_weight)
    return flat.reshape(B, OUTW)
